# baseline (speedup 1.0000x reference)
import functools

import jax
import jax.numpy as jnp
from jax.experimental import pallas as pl
from jax.experimental.pallas import tpu as pltpu

_B, _N, _S, _H = 4, 512, 64, 32
_PACK = 4
_KP = _PACK * _S
_HP = _PACK * _H
_JP = _N // _PACK
_CH = 8


def _body(nm_ref, x_ref, w_ref, b_ref, o_ref):
    g = pl.program_id(0)
    x = x_ref[...].reshape(_CH * _JP, _KP)
    y = jnp.dot(x, w_ref[...], preferred_element_type=jnp.float32)
    y = y + b_ref[0]
    y = y.reshape(_CH, _JP, _HP)
    for c in range(_CH):
        o_ref[c] = y[c] * nm_ref[g * _CH + c]


@jax.jit
def kernel(stacks, mask, W, b_lin):
    x = stacks.reshape(_B * _N, _JP, _KP)
    wt = W.T.astype(jnp.float32)
    wp = jax.scipy.linalg.block_diag(*([wt] * _PACK))
    bp = jnp.tile(b_lin, _PACK).reshape(1, _HP)
    nm = 1.0 - mask.reshape(-1).astype(jnp.float32)

    grid = (_B * _N // _CH,)
    out = pl.pallas_call(
        _body,
        grid_spec=pltpu.PrefetchScalarGridSpec(
            num_scalar_prefetch=1,
            grid=grid,
            in_specs=[
                pl.BlockSpec((_CH, _JP, _KP), lambda g, nm_ref: (g, 0, 0)),
                pl.BlockSpec((_KP, _HP), lambda g, nm_ref: (0, 0)),
                pl.BlockSpec((1, _HP), lambda g, nm_ref: (0, 0)),
            ],
            out_specs=pl.BlockSpec((_CH, _JP, _HP), lambda g, nm_ref: (g, 0, 0)),
        ),
        out_shape=jax.ShapeDtypeStruct((_B * _N, _JP, _HP), jnp.float32),
        compiler_params=pltpu.CompilerParams(
            dimension_semantics=("arbitrary",),
        ),
    )(nm, x, wp, bp)
    return out.reshape(_B, _N, _N, _H)


# speedup vs baseline: 1.2041x; 1.2041x over previous
import jax
import jax.numpy as jnp
from jax.experimental import pallas as pl
from jax.experimental.pallas import tpu as pltpu

_B, _N, _S, _H = 4, 512, 64, 32
_CH = 8


def _body(nm_ref, x_ref, w_ref, b_ref, o_ref):
    b = pl.program_id(0)
    ib = pl.program_id(1)
    x = x_ref[0].reshape(_CH * _N, _S)
    y = jnp.dot(x, w_ref[...], preferred_element_type=jnp.float32)
    y = (y + b_ref[0]).reshape(_CH, _N, _H)
    base = b * _N + ib * _CH
    for c in range(_CH):
        o_ref[0, c] = y[c] * nm_ref[base + c]


@jax.jit
def kernel(stacks, mask, W, b_lin):
    wt = W.T.astype(jnp.float32)
    bp = jnp.broadcast_to(b_lin.reshape(1, _H), (8, _H))
    nm = 1.0 - mask.reshape(-1).astype(jnp.float32)

    grid = (_B, _N // _CH)
    out = pl.pallas_call(
        _body,
        grid_spec=pltpu.PrefetchScalarGridSpec(
            num_scalar_prefetch=1,
            grid=grid,
            in_specs=[
                pl.BlockSpec((1, _CH, _N, _S), lambda b, ib, nm_ref: (b, ib, 0, 0)),
                pl.BlockSpec((_S, _H), lambda b, ib, nm_ref: (0, 0)),
                pl.BlockSpec((8, _H), lambda b, ib, nm_ref: (0, 0)),
            ],
            out_specs=pl.BlockSpec(
                (1, _CH, _N, _H), lambda b, ib, nm_ref: (b, ib, 0, 0)
            ),
        ),
        out_shape=jax.ShapeDtypeStruct((_B, _N, _N, _H), jnp.float32),
        compiler_params=pltpu.CompilerParams(
            dimension_semantics=("arbitrary", "arbitrary"),
        ),
    )(nm, stacks, wt, bp)
    return out


# speedup vs baseline: 1.2675x; 1.0526x over previous
import jax
import jax.numpy as jnp
from jax.experimental import pallas as pl
from jax.experimental.pallas import tpu as pltpu

_B, _N, _S, _H = 4, 512, 64, 32
_CH = 32


def _body(nm_ref, x_ref, w_ref, b_ref, o_ref):
    b = pl.program_id(0)
    ib = pl.program_id(1)
    x = x_ref[0].reshape(_CH * _N, _S)
    y = jnp.dot(x, w_ref[...], preferred_element_type=jnp.float32)
    y = (y + b_ref[0]).reshape(_CH, _N, _H)
    base = b * _N + ib * _CH
    for c in range(_CH):
        o_ref[0, c] = y[c] * nm_ref[base + c]


@jax.jit
def kernel(stacks, mask, W, b_lin):
    wt = W.T.astype(jnp.float32)
    bp = jnp.broadcast_to(b_lin.reshape(1, _H), (8, _H))
    nm = 1.0 - mask.reshape(-1).astype(jnp.float32)

    grid = (_B, _N // _CH)
    out = pl.pallas_call(
        _body,
        grid_spec=pltpu.PrefetchScalarGridSpec(
            num_scalar_prefetch=1,
            grid=grid,
            in_specs=[
                pl.BlockSpec((1, _CH, _N, _S), lambda b, ib, nm_ref: (b, ib, 0, 0)),
                pl.BlockSpec((_S, _H), lambda b, ib, nm_ref: (0, 0)),
                pl.BlockSpec((8, _H), lambda b, ib, nm_ref: (0, 0)),
            ],
            out_specs=pl.BlockSpec(
                (1, _CH, _N, _H), lambda b, ib, nm_ref: (b, ib, 0, 0)
            ),
        ),
        out_shape=jax.ShapeDtypeStruct((_B, _N, _N, _H), jnp.float32),
        compiler_params=pltpu.CompilerParams(
            dimension_semantics=("arbitrary", "arbitrary"),
        ),
    )(nm, stacks, wt, bp)
    return out
